# Initial kernel scaffold; baseline (speedup 1.0000x reference)
import jax
import jax.numpy as jnp
from jax.experimental import pallas as pl


def kernel(x, skip, w1, b1, g1, be1, w2, b2, g2, be2, wsx, wss, bs):
    raise NotImplementedError("write your pallas kernel here")


# speedup vs baseline: 1.1471x; 1.1471x over previous
from functools import partial

import numpy as np
import jax
import jax.numpy as jnp
from jax import lax
from jax.experimental import pallas as pl
from jax.experimental.pallas import tpu as pltpu

_EPS = 1e-5
_INV_SQRT2 = 0.7071067811865475


def _gelu_exact(v):
    return 0.5 * v * (1.0 + lax.erf(v * _INV_SQRT2))


def _tap_conv(slab_ref, wtaps, ml, mr, *, width, margin, pix):
    acc = None
    for tap in range(9):
        dy, dx = tap // 3 - 1, tap % 3 - 1
        lo = margin + dy * width + dx
        piece = slab_ref[:, lo:lo + pix]
        if dx == -1:
            piece = piece * ml
        elif dx == 1:
            piece = piece * mr
        term = jnp.dot(wtaps[tap], piece, preferred_element_type=jnp.float32)
        acc = term if acc is None else acc + term
    return acc


def _stage1(x_ref, skip_ref, mup_ref, edge_ref, w1_ref, b1_ref, ws_ref,
            bs_ref, y1_ref, s1_ref, q1_ref, ys_ref, slab_ref,
            *, width, margin):
    c2, p4 = x_ref.shape[1], x_ref.shape[2]
    d, p = skip_ref.shape[1], skip_ref.shape[2]
    c3, m = c2 + d, margin

    z = jnp.zeros((c3, m), jnp.bfloat16)
    slab_ref[:, 0:m] = z
    slab_ref[:, m + p:m + p + m] = z

    up = jnp.dot(x_ref[...].reshape(c2, p4), mup_ref[...],
                 preferred_element_type=jnp.float32)
    slab_ref[0:c2, m:m + p] = up.astype(jnp.bfloat16)
    slab_ref[c2:c3, m:m + p] = skip_ref[...].reshape(d, p)

    raw = _tap_conv(slab_ref, w1_ref[...], edge_ref[0:1, :], edge_ref[1:2, :],
                    width=width, margin=m, pix=p) + b1_ref[...]

    ys = jnp.dot(ws_ref[...], slab_ref[:, m:m + p],
                 preferred_element_type=jnp.float32) + bs_ref[...]

    y1_ref[...] = raw.reshape(1, d, p).astype(y1_ref.dtype)
    ys_ref[...] = ys.reshape(1, d, p).astype(ys_ref.dtype)
    s1_ref[...] = jnp.sum(raw, axis=1).reshape(1, d, 1)
    q1_ref[...] = jnp.sum(raw * raw, axis=1).reshape(1, d, 1)


def _stage2(y1_ref, sc1_ref, sh1_ref, edge_ref, w2_ref, b2_ref,
            y2_ref, s2_ref, q2_ref, slab_ref, *, width, margin):
    d, p = y1_ref.shape[1], y1_ref.shape[2]
    m = margin

    z = jnp.zeros((d, m), jnp.bfloat16)
    slab_ref[:, 0:m] = z
    slab_ref[:, m + p:m + p + m] = z

    act = _gelu_exact(y1_ref[...].reshape(d, p).astype(jnp.float32)
                      * sc1_ref[...] + sh1_ref[...])
    slab_ref[:, m:m + p] = act.astype(jnp.bfloat16)

    raw = _tap_conv(slab_ref, w2_ref[...], edge_ref[0:1, :], edge_ref[1:2, :],
                    width=width, margin=m, pix=p) + b2_ref[...]

    y2_ref[...] = raw.reshape(1, d, p).astype(y2_ref.dtype)
    s2_ref[...] = jnp.sum(raw, axis=1).reshape(1, d, 1)
    q2_ref[...] = jnp.sum(raw * raw, axis=1).reshape(1, d, 1)


def _stage3(y2_ref, sc2_ref, sh2_ref, ys_ref, out_ref):
    d, p = y2_ref.shape[1], y2_ref.shape[2]
    act = _gelu_exact(y2_ref[...].reshape(d, p).astype(jnp.float32)
                      * sc2_ref[...] + sh2_ref[...])
    out_ref[...] = (act + ys_ref[...].reshape(d, p).astype(jnp.float32)
                    ).reshape(1, d, p)


def _finalize_bn(s, q, gamma, beta, count):
    tot = jnp.sum(s[:, :, 0], axis=0)
    totsq = jnp.sum(q[:, :, 0], axis=0)
    mu = tot / count
    var = totsq / count - mu * mu
    inv = lax.rsqrt(jnp.maximum(var, 0.0) + _EPS)
    sc = gamma * inv
    sh = beta - mu * sc
    d = sc.shape[0]
    return sc.reshape(d, 1), sh.reshape(d, 1)


def _params(sems):
    return pltpu.CompilerParams(dimension_semantics=sems,
                                vmem_limit_bytes=100 * 1024 * 1024)


def kernel(x, skip, w1, b1, g1, be1, w2, b2, g2, be2, wsx, wss, bs):
    n, c2, hh, ww = x.shape
    _, d, hgt, wid = skip.shape
    c3 = c2 + d
    p4, p = hh * ww, hgt * wid
    m = max(128, pl.cdiv(wid + 1, 128) * 128)
    bf16, f32 = jnp.bfloat16, jnp.float32

    xb = x.reshape(n, c2, p4).astype(bf16)
    sb = skip.reshape(n, d, p).astype(bf16)
    w1b = w1.astype(bf16)
    w2b = w2.astype(bf16)
    wsb = jnp.concatenate([wsx, wss], axis=1).astype(bf16)

    src = ((np.arange(hgt)[:, None] // 2) * ww
           + (np.arange(wid)[None, :] // 2)).reshape(-1)
    mup = jnp.asarray(np.arange(p4)[:, None] == src[None, :], bf16)
    col = np.arange(p) % wid
    edge = jnp.asarray(np.stack([col != 0, col != wid - 1]), bf16)

    y1, s1, q1, ys = pl.pallas_call(
        partial(_stage1, width=wid, margin=m),
        grid=(n,),
        in_specs=[
            pl.BlockSpec((1, c2, p4), lambda i: (i, 0, 0)),
            pl.BlockSpec((1, d, p), lambda i: (i, 0, 0)),
            pl.BlockSpec((p4, p), lambda i: (0, 0)),
            pl.BlockSpec((2, p), lambda i: (0, 0)),
            pl.BlockSpec((9, d, c3), lambda i: (0, 0, 0)),
            pl.BlockSpec((d, 1), lambda i: (0, 0)),
            pl.BlockSpec((d, c3), lambda i: (0, 0)),
            pl.BlockSpec((d, 1), lambda i: (0, 0)),
        ],
        out_specs=(
            pl.BlockSpec((1, d, p), lambda i: (i, 0, 0)),
            pl.BlockSpec((1, d, 1), lambda i: (i, 0, 0)),
            pl.BlockSpec((1, d, 1), lambda i: (i, 0, 0)),
            pl.BlockSpec((1, d, p), lambda i: (i, 0, 0)),
        ),
        out_shape=(
            jax.ShapeDtypeStruct((n, d, p), bf16),
            jax.ShapeDtypeStruct((n, d, 1), f32),
            jax.ShapeDtypeStruct((n, d, 1), f32),
            jax.ShapeDtypeStruct((n, d, p), bf16),
        ),
        scratch_shapes=[pltpu.VMEM((c3, 2 * m + p), bf16)],
        compiler_params=_params(("parallel",)),
    )(xb, sb, mup, edge, w1b, b1, wsb, bs)

    sc1, sh1 = _finalize_bn(s1, q1, g1, be1, float(n * p))

    y2, s2, q2 = pl.pallas_call(
        partial(_stage2, width=wid, margin=m),
        grid=(n,),
        in_specs=[
            pl.BlockSpec((1, d, p), lambda i: (i, 0, 0)),
            pl.BlockSpec((d, 1), lambda i: (0, 0)),
            pl.BlockSpec((d, 1), lambda i: (0, 0)),
            pl.BlockSpec((2, p), lambda i: (0, 0)),
            pl.BlockSpec((9, d, d), lambda i: (0, 0, 0)),
            pl.BlockSpec((d, 1), lambda i: (0, 0)),
        ],
        out_specs=(
            pl.BlockSpec((1, d, p), lambda i: (i, 0, 0)),
            pl.BlockSpec((1, d, 1), lambda i: (i, 0, 0)),
            pl.BlockSpec((1, d, 1), lambda i: (i, 0, 0)),
        ),
        out_shape=(
            jax.ShapeDtypeStruct((n, d, p), bf16),
            jax.ShapeDtypeStruct((n, d, 1), f32),
            jax.ShapeDtypeStruct((n, d, 1), f32),
        ),
        scratch_shapes=[pltpu.VMEM((d, 2 * m + p), bf16)],
        compiler_params=_params(("parallel",)),
    )(y1, sc1, sh1, edge, w2b, b2)

    sc2, sh2 = _finalize_bn(s2, q2, g2, be2, float(n * p))

    out = pl.pallas_call(
        _stage3,
        grid=(n,),
        in_specs=[
            pl.BlockSpec((1, d, p), lambda i: (i, 0, 0)),
            pl.BlockSpec((d, 1), lambda i: (0, 0)),
            pl.BlockSpec((d, 1), lambda i: (0, 0)),
            pl.BlockSpec((1, d, p), lambda i: (i, 0, 0)),
        ],
        out_specs=pl.BlockSpec((1, d, p), lambda i: (i, 0, 0)),
        out_shape=jax.ShapeDtypeStruct((n, d, p), f32),
        compiler_params=_params(("parallel",)),
    )(y2, sc2, sh2, ys)

    return out.reshape(n, d, hgt, wid)
